# baseline (speedup 1.0000x reference)
import functools

import jax
import jax.numpy as jnp
from jax.experimental import pallas as pl

DROP_P = 0.1
KEY_LO = 42
KEY_HI = 0
T = 2048

_ROT_A = (13, 15, 26, 6)
_ROT_B = (17, 29, 16, 24)


def _rotl(x, r):
    return (x << jnp.uint32(r)) | (x >> jnp.uint32(32 - r))


def _threefry2x32(x0, x1):
    k0 = jnp.uint32(KEY_HI)
    k1 = jnp.uint32(KEY_LO)
    ks = (k0, k1, k0 ^ k1 ^ jnp.uint32(0x1BD11BDA))
    x0 = x0 + ks[0]
    x1 = x1 + ks[1]
    for i in range(5):
        for r in (_ROT_A, _ROT_B)[i % 2]:
            x0 = x0 + x1
            x1 = _rotl(x1, r)
            x1 = x1 ^ x0
        x0 = x0 + ks[(i + 1) % 3]
        x1 = x1 + ks[(i + 2) % 3] + jnp.uint32(i + 1)
    return x0, x1


def _mask_body(o_ref):
    rows, lanes = o_ref.shape
    sub = jax.lax.broadcasted_iota(jnp.uint32, (rows, lanes), 0)
    lane = jax.lax.broadcasted_iota(jnp.uint32, (rows, lanes), 1)
    i_global = sub * jnp.uint32(lanes) + lane
    a, b = _threefry2x32(jnp.zeros_like(i_global), i_global)
    bits = a ^ b
    u = jax.lax.bitcast_convert_type(
        (bits >> jnp.uint32(9)) | jnp.uint32(0x3F800000), jnp.float32
    ) - jnp.float32(1.0)
    dropped = u < jnp.float32(DROP_P)
    first_t = (i_global % jnp.uint32(T)) == jnp.uint32(0)
    keep = (~dropped) | first_t
    o_ref[...] = jnp.where(keep, jnp.float32(1.0), jnp.float32(0.0))


def _mul_body(x_ref, m_ref, o_ref):
    o_ref[...] = x_ref[...] * m_ref[...]


def kernel(x):
    B, t, F = x.shape
    rows = B * t
    xr = x.reshape(rows // 128, 128, F)

    mul2d = pl.pallas_call(
        _mask_body,
        out_shape=jax.ShapeDtypeStruct((rows // 128, 128), jnp.float32),
    )()
    mul = mul2d.reshape(rows // 128, 128, 1)

    rpb = 4
    grid = (xr.shape[0] // rpb,)
    out = pl.pallas_call(
        _mul_body,
        grid=grid,
        in_specs=[
            pl.BlockSpec((rpb, 128, F), lambda i: (i, 0, 0)),
            pl.BlockSpec((rpb, 128, 1), lambda i: (i, 0, 0)),
        ],
        out_specs=pl.BlockSpec((rpb, 128, F), lambda i: (i, 0, 0)),
        out_shape=jax.ShapeDtypeStruct(xr.shape, x.dtype),
    )(xr, mul)
    return out.reshape(B, t, F)


# speedup vs baseline: 1.0572x; 1.0572x over previous
import functools

import jax
import jax.numpy as jnp
from jax.experimental import pallas as pl
from jax.experimental.pallas import tpu as pltpu

DROP_P = 0.1
KEY_LO = 42
KEY_HI = 0
T = 2048

_ROT_A = (13, 15, 26, 6)
_ROT_B = (17, 29, 16, 24)


def _rotl(x, r):
    return (x << jnp.uint32(r)) | (x >> jnp.uint32(32 - r))


def _threefry2x32(x0, x1):
    k0 = jnp.uint32(KEY_HI)
    k1 = jnp.uint32(KEY_LO)
    ks = (k0, k1, k0 ^ k1 ^ jnp.uint32(0x1BD11BDA))
    x0 = x0 + ks[0]
    x1 = x1 + ks[1]
    for i in range(5):
        for r in (_ROT_A, _ROT_B)[i % 2]:
            x0 = x0 + x1
            x1 = _rotl(x1, r)
            x1 = x1 ^ x0
        x0 = x0 + ks[(i + 1) % 3]
        x1 = x1 + ks[(i + 2) % 3] + jnp.uint32(i + 1)
    return x0, x1


def _keep_multiplier(rows, lanes):
    sub = jax.lax.broadcasted_iota(jnp.uint32, (rows, lanes), 0)
    lane = jax.lax.broadcasted_iota(jnp.uint32, (rows, lanes), 1)
    i_global = sub * jnp.uint32(lanes) + lane
    a, b = _threefry2x32(jnp.zeros_like(i_global), i_global)
    bits = a ^ b
    u = jax.lax.bitcast_convert_type(
        (bits >> jnp.uint32(9)) | jnp.uint32(0x3F800000), jnp.float32
    ) - jnp.float32(1.0)
    dropped = u < jnp.float32(DROP_P)
    first_t = (i_global % jnp.uint32(T)) == jnp.uint32(0)
    keep = (~dropped) | first_t
    return jnp.where(keep, jnp.float32(1.0), jnp.float32(0.0))


def _body(rpb, x_ref, o_ref, m_scr):
    @pl.when(pl.program_id(0) == 0)
    def _():
        m_scr[:, :, 0] = _keep_multiplier(*m_scr.shape[:2])

    base = pl.program_id(0) * rpb
    o_ref[...] = x_ref[...] * m_scr[pl.ds(base, rpb), :, :]


def kernel(x):
    B, t, F = x.shape
    rows = B * t
    xr = x.reshape(rows // 128, 128, F)
    rpb = 4
    grid = (xr.shape[0] // rpb,)
    out = pl.pallas_call(
        functools.partial(_body, rpb),
        grid=grid,
        in_specs=[pl.BlockSpec((rpb, 128, F), lambda i: (i, 0, 0))],
        out_specs=pl.BlockSpec((rpb, 128, F), lambda i: (i, 0, 0)),
        out_shape=jax.ShapeDtypeStruct(xr.shape, x.dtype),
        scratch_shapes=[pltpu.VMEM((rows // 128, 128, 1), jnp.float32)],
    )(xr)
    return out.reshape(B, t, F)
